# baseline (speedup 1.0000x reference)
import functools

import jax
import jax.numpy as jnp
from jax import lax
from jax.experimental import pallas as pl
from jax.experimental.pallas import tpu as pltpu
from jax.experimental.pallas import tpu_sc as plsc

_NUM_NODES = 1000000
_EMBED_DIM = 32
_BATCH = 16384

_info = plsc.get_sparse_core_info()
_NC, _NS = _info.num_cores, _info.num_subcores
_NW = _NC * _NS
_B_PER_W = _BATCH // _NW


def _gather_body(emb_hbm, idx_hbm, out_hbm, idx_v, rows_v, sem):
    wid = lax.axis_index("s") * _NC + lax.axis_index("c")
    base = wid * _B_PER_W
    pltpu.sync_copy(idx_hbm.at[pl.ds(base, _B_PER_W)], idx_v)
    pltpu.async_copy(emb_hbm.at[idx_v], rows_v, sem).wait()
    pltpu.sync_copy(rows_v, out_hbm.at[pl.ds(base, _B_PER_W)])


@jax.jit
def kernel(batch, emb):
    idx = batch.astype(jnp.int32)
    mesh = plsc.VectorSubcoreMesh(core_axis_name="c", subcore_axis_name="s")
    gather = pl.kernel(
        _gather_body,
        out_type=jax.ShapeDtypeStruct((_BATCH, _EMBED_DIM), jnp.float32),
        mesh=mesh,
        scratch_types=[
            pltpu.VMEM((_B_PER_W,), jnp.int32),
            pltpu.VMEM((_B_PER_W, _EMBED_DIM), jnp.float32),
            pltpu.SemaphoreType.DMA,
        ],
        compiler_params=pltpu.CompilerParams(use_tc_tiling_on_sc=False),
    )
    return gather(emb, idx)


# speedup vs baseline: 4.4172x; 4.4172x over previous
import jax
import jax.numpy as jnp
from jax import lax
from jax.experimental import pallas as pl
from jax.experimental.pallas import tpu as pltpu
from jax.experimental.pallas import tpu_sc as plsc

_NUM_NODES = 1000000
_EMBED_DIM = 32
_BATCH = 16384
_LANES = 128
_OUT_W = 128

_info = plsc.get_sparse_core_info()
_NC, _NS = _info.num_cores, _info.num_subcores
_NW = _NC * _NS
_B_PER_W = _BATCH // _NW
_WAVE = 8
_N_WAVES = _B_PER_W // _WAVE


def _gather_body(embT_hbm, idx_hbm, out_hbm, idx_v, tiles_v, stg_v,
                 sem_f, sem_o):
    wid = lax.axis_index("s") * _NC + lax.axis_index("c")
    base = wid * _B_PER_W

    pltpu.sync_copy(idx_hbm.at[pl.ds(base, _B_PER_W)], idx_v.at[pl.ds(0, _B_PER_W)])

    def fire(w):
        buf = lax.rem(w, 2)
        vecw = idx_v[pl.ds(w * _WAVE, 16)]
        for j in range(_WAVE):
            n = vecw[j]
            grp = pl.multiple_of((n // _LANES) * _LANES, _LANES)
            pltpu.async_copy(
                embT_hbm.at[:, pl.ds(grp, _LANES)],
                tiles_v.at[buf * _WAVE + j],
                sem_f,
            )

    fire(jnp.int32(0))

    k16 = lax.iota(jnp.int32, 16)
    jv = lax.rem(k16, _WAVE)
    fh = k16 // _WAVE

    def wave(w, carry):
        buf = lax.rem(w, 2)
        @pl.when(w < _N_WAVES - 1)
        def _():
            fire(w + 1)

        for j in range(_WAVE):
            pltpu.make_async_copy(
                embT_hbm.at[:, pl.ds(0, _LANES)],
                tiles_v.at[buf * _WAVE + j],
                sem_f,
            ).wait()

        @pl.when(w >= 2)
        def _():
            pltpu.make_async_copy(
                stg_v.at[buf], out_hbm.at[pl.ds(0, _WAVE), :], sem_o
            ).wait()

        nvec = plsc.load_gather(idx_v, [w * _WAVE + jv])
        lanes = lax.rem(nvec, _LANES)
        slot = buf * _WAVE + jv
        for fg in range(_EMBED_DIM // 2):
            fvec = fh + 2 * fg
            vals = plsc.load_gather(tiles_v, [slot, fvec, lanes])
            plsc.store_scatter(stg_v.at[buf], [jv, fvec], vals)

        row0 = pl.multiple_of(base + w * _WAVE, _WAVE)
        pltpu.async_copy(
            stg_v.at[buf],
            out_hbm.at[pl.ds(row0, _WAVE), :],
            sem_o,
        )
        return carry

    lax.fori_loop(0, _N_WAVES, wave, jnp.int32(0))

    for _ in range(2):
        pltpu.make_async_copy(
            stg_v.at[0], out_hbm.at[pl.ds(0, _WAVE), :], sem_o
        ).wait()


@jax.jit
def kernel(batch, emb):
    idx = batch.astype(jnp.int32)
    mesh = plsc.VectorSubcoreMesh(core_axis_name="c", subcore_axis_name="s")
    gather = pl.kernel(
        _gather_body,
        out_type=jax.ShapeDtypeStruct((_BATCH, _OUT_W), jnp.float32),
        mesh=mesh,
        scratch_types=[
            pltpu.VMEM((_B_PER_W + _WAVE,), jnp.int32),
            pltpu.VMEM((2 * _WAVE, _EMBED_DIM, _LANES), jnp.float32),
            pltpu.VMEM((2, _WAVE, _OUT_W), jnp.float32),
            pltpu.SemaphoreType.DMA,
            pltpu.SemaphoreType.DMA,
        ],
        compiler_params=pltpu.CompilerParams(
            use_tc_tiling_on_sc=True, needs_layout_passes=False
        ),
    )
    out_wide = gather(emb.T, idx)
    return out_wide[:, :_EMBED_DIM]


# speedup vs baseline: 4.6315x; 1.0485x over previous
import jax
import jax.numpy as jnp
from jax import lax
from jax.experimental import pallas as pl
from jax.experimental.pallas import tpu as pltpu
from jax.experimental.pallas import tpu_sc as plsc

_NUM_NODES = 1000000
_EMBED_DIM = 32
_BATCH = 16384
_LANES = 128

_info = plsc.get_sparse_core_info()
_NC, _NS = _info.num_cores, _info.num_subcores
_NW = _NC * _NS
_B_PER_W = _BATCH // _NW
_WAVE = 8
_N_WAVES = _B_PER_W // _WAVE
_RING = 3


def _gather_body(embT_hbm, idx_hbm, outT_hbm, idx_v, tiles_v, blk_v, sem_f):
    wid = lax.axis_index("s") * _NC + lax.axis_index("c")
    base = wid * _B_PER_W

    pltpu.sync_copy(idx_hbm.at[pl.ds(base, _B_PER_W)], idx_v.at[pl.ds(0, _B_PER_W)])

    def fire(w):
        buf = lax.rem(w, _RING)
        vecw = idx_v[pl.ds(w * _WAVE, 16)]
        for j in range(_WAVE):
            n = vecw[j]
            grp = pl.multiple_of((n // _LANES) * _LANES, _LANES)
            pltpu.async_copy(
                embT_hbm.at[:, pl.ds(grp, _LANES)],
                tiles_v.at[buf * _WAVE + j],
                sem_f,
            )

    fire(jnp.int32(0))
    fire(jnp.int32(1))

    k16 = lax.iota(jnp.int32, 16)
    jv = lax.rem(k16, _WAVE)
    fh = k16 // _WAVE

    def wave(w, carry):
        buf = lax.rem(w, _RING)
        @pl.when(w < _N_WAVES - 2)
        def _():
            fire(w + 2)

        for j in range(_WAVE):
            pltpu.make_async_copy(
                embT_hbm.at[:, pl.ds(0, _LANES)],
                tiles_v.at[buf * _WAVE + j],
                sem_f,
            ).wait()

        nvec = plsc.load_gather(idx_v, [w * _WAVE + jv])
        lanes = lax.rem(nvec, _LANES)
        slot = buf * _WAVE + jv
        col = w * _WAVE + jv
        for fg in range(_EMBED_DIM // 2):
            fvec = fh + 2 * fg
            vals = plsc.load_gather(tiles_v, [slot, fvec, lanes])
            plsc.store_scatter(blk_v, [fvec, col], vals)
        return carry

    lax.fori_loop(0, _N_WAVES, wave, jnp.int32(0))

    col0 = pl.multiple_of(base, _LANES)
    pltpu.sync_copy(blk_v, outT_hbm.at[:, pl.ds(col0, _B_PER_W)])


@jax.jit
def kernel(batch, emb):
    idx = batch.astype(jnp.int32)
    mesh = plsc.VectorSubcoreMesh(core_axis_name="c", subcore_axis_name="s")
    gather = pl.kernel(
        _gather_body,
        out_type=jax.ShapeDtypeStruct((_EMBED_DIM, _BATCH), jnp.float32),
        mesh=mesh,
        scratch_types=[
            pltpu.VMEM((_B_PER_W + _WAVE,), jnp.int32),
            pltpu.VMEM((_RING * _WAVE, _EMBED_DIM, _LANES), jnp.float32),
            pltpu.VMEM((_EMBED_DIM, _B_PER_W), jnp.float32),
            pltpu.SemaphoreType.DMA,
        ],
        compiler_params=pltpu.CompilerParams(
            use_tc_tiling_on_sc=True, needs_layout_passes=False
        ),
    )
    return gather(emb.T, idx).T
